# Initial kernel scaffold; baseline (speedup 1.0000x reference)
import jax
import jax.numpy as jnp
from jax.experimental import pallas as pl


def kernel(x_skill, x_job, ei_req, ew_req, ei_rev_req, ew_rev_req, ei_ss, ew_ss, ei_rev_ss, ew_rev_ss, ei_jj, ew_jj, ei_rev_jj, ew_rev_jj, num_sampled_nodes, num_sampled_edges, W_in_s, b_in_s, W_in_j, b_in_j, req_Wmp, req_bmp, req_Wj, req_bj, req_Wi, req_bi, rev_req_Wmp, rev_req_bmp, rev_req_Wj, rev_req_bj, rev_req_Wi, rev_req_bi, ss_Wmp, ss_bmp, ss_Wj, ss_bj, ss_Wi, ss_bi, jj_Wmp, jj_bmp, jj_Wj, jj_bj, jj_Wi, jj_bi, W_out_s, b_out_s, W_out_j, b_out_j):
    raise NotImplementedError("write your pallas kernel here")


# speedup vs baseline: 1.1943x; 1.1943x over previous
import functools

import jax
import jax.numpy as jnp
from jax import lax
from jax.experimental import pallas as pl
from jax.experimental.pallas import tpu as pltpu
from jax.experimental.pallas import tpu_sc as plsc

NS = 50000
D = 128
H = 128
HALF = 64
BLK = 2000


def _stage1_body(x_ref, Win, bin_, Wa, ba, Wb, bb, Wia, bia, Wib, bib,
                 ha_ref, hb_ref, xda_ref, xdb_ref):
    x = x_ref[...]
    h = jnp.maximum(
        jnp.dot(x, Win[...], preferred_element_type=jnp.float32) + bin_[...], 0.0)
    ha_ref[...] = jnp.maximum(
        jnp.dot(h, Wa[...], preferred_element_type=jnp.float32) + ba[...], 0.0)
    hb_ref[...] = jnp.maximum(
        jnp.dot(h, Wb[...], preferred_element_type=jnp.float32) + bb[...], 0.0)
    xda_ref[...] = jnp.dot(h, Wia[...], preferred_element_type=jnp.float32) + bia[...]
    xdb_ref[...] = jnp.dot(h, Wib[...], preferred_element_type=jnp.float32) + bib[...]


def _stage1(x, Win, bin_, Wa, ba, Wb, bb, Wia, bia, Wib, bib):
    n = x.shape[0]
    grid = n // BLK
    row = pl.BlockSpec((BLK, H), lambda i: (i, 0))
    w_full = pl.BlockSpec((H, H), lambda i: (0, 0))
    w_half = pl.BlockSpec((H, HALF), lambda i: (0, 0))
    b_full = pl.BlockSpec((1, H), lambda i: (0, 0))
    b_half = pl.BlockSpec((1, HALF), lambda i: (0, 0))
    rowh = pl.BlockSpec((BLK, HALF), lambda i: (i, 0))
    return pl.pallas_call(
        _stage1_body,
        grid=(grid,),
        in_specs=[row, w_full, b_full, w_full, b_full, w_full, b_full,
                  w_half, b_half, w_half, b_half],
        out_specs=[row, row, rowh, rowh],
        out_shape=[
            jax.ShapeDtypeStruct((n, H), jnp.float32),
            jax.ShapeDtypeStruct((n, H), jnp.float32),
            jax.ShapeDtypeStruct((n, HALF), jnp.float32),
            jax.ShapeDtypeStruct((n, HALF), jnp.float32),
        ],
    )(x, Win, bin_.reshape(1, H), Wa, ba.reshape(1, H), Wb, bb.reshape(1, H),
      Wia, bia.reshape(1, HALF), Wib, bib.reshape(1, HALF))


def _stage3_body(xd1, xd2, xd3, a1, a2, a3, Wj1, bj1, Wj2, bj2, Wj3, bj3,
                 Wo, bo, out_ref):
    s = jnp.zeros((BLK, H), jnp.float32)
    for xd, a, Wj, bj in ((xd1, a1, Wj1, bj1), (xd2, a2, Wj2, bj2),
                          (xd3, a3, Wj3, bj3)):
        t = jnp.dot(a[...], Wj[...], preferred_element_type=jnp.float32) + bj[...]
        u = jnp.maximum(jnp.concatenate([xd[...], t], axis=-1), 0.0)
        nrm = jnp.maximum(jnp.sqrt(jnp.sum(u * u, axis=-1, keepdims=True)), 1e-12)
        s = s + u / nrm
    out_ref[...] = jnp.maximum(
        jnp.dot(s, Wo[...], preferred_element_type=jnp.float32) + bo[...], 0.0)


def _stage3(xd1, xd2, xd3, a1, a2, a3, Wj1, bj1, Wj2, bj2, Wj3, bj3, Wo, bo):
    n = NS
    grid = n // BLK
    row = pl.BlockSpec((BLK, H), lambda i: (i, 0))
    rowh = pl.BlockSpec((BLK, HALF), lambda i: (i, 0))
    w_half = pl.BlockSpec((H, HALF), lambda i: (0, 0))
    b_half = pl.BlockSpec((1, HALF), lambda i: (0, 0))
    w_full = pl.BlockSpec((H, H), lambda i: (0, 0))
    b_full = pl.BlockSpec((1, H), lambda i: (0, 0))
    return pl.pallas_call(
        _stage3_body,
        grid=(grid,),
        in_specs=[rowh, rowh, rowh, row, row, row,
                  w_half, b_half, w_half, b_half, w_half, b_half,
                  w_full, b_full],
        out_specs=row,
        out_shape=jax.ShapeDtypeStruct((n, H), jnp.float32),
    )(xd1, xd2, xd3, a1[:n], a2[:n], a3[:n],
      Wj1, bj1.reshape(1, HALF), Wj2, bj2.reshape(1, HALF),
      Wj3, bj3.reshape(1, HALF), Wo, bo.reshape(1, H))


def _seg_max(h, src, dst, ew, n_dst):
    msg = h[src] * ew[:, None]
    return jax.ops.segment_max(msg, dst, num_segments=n_dst,
                               indices_are_sorted=False)


def _seg_max_fix(agg):
    return jnp.where(jnp.isfinite(agg), agg, 0.0)


def kernel(x_skill, x_job, ei_req, ew_req, ei_rev_req, ew_rev_req, ei_ss, ew_ss, ei_rev_ss, ew_rev_ss, ei_jj, ew_jj, ei_rev_jj, ew_rev_jj, num_sampled_nodes, num_sampled_edges, W_in_s, b_in_s, W_in_j, b_in_j, req_Wmp, req_bmp, req_Wj, req_bj, req_Wi, req_bi, rev_req_Wmp, rev_req_bmp, rev_req_Wj, rev_req_bj, rev_req_Wi, rev_req_bi, ss_Wmp, ss_bmp, ss_Wj, ss_bj, ss_Wi, ss_bi, jj_Wmp, jj_bmp, jj_Wj, jj_bj, jj_Wi, jj_bi, W_out_s, b_out_s, W_out_j, b_out_j):
    h_revreq, h_ss, xd_req, xd_ss = _stage1(
        x_skill, W_in_s, b_in_s, rev_req_Wmp, rev_req_bmp, ss_Wmp, ss_bmp,
        req_Wi, req_bi, ss_Wi, ss_bi)
    h_req, h_jj, xd_revreq, xd_jj = _stage1(
        x_job, W_in_j, b_in_j, req_Wmp, req_bmp, jj_Wmp, jj_bmp,
        rev_req_Wi, rev_req_bi, jj_Wi, jj_bi)

    agg_req = _seg_max_fix(_seg_max(h_req, ei_req[0], ei_req[1], ew_req, NS))
    agg_ss = _seg_max_fix(_seg_max(h_ss, ei_ss[0], ei_ss[1], ew_ss, NS))
    agg_rss = _seg_max_fix(_seg_max(h_ss, ei_rev_ss[0], ei_rev_ss[1], ew_rev_ss, NS))
    agg_rreq = _seg_max_fix(_seg_max(h_revreq, ei_rev_req[0], ei_rev_req[1], ew_rev_req, NS))
    agg_jj = _seg_max_fix(_seg_max(h_jj, ei_jj[0], ei_jj[1], ew_jj, NS))
    agg_rjj = _seg_max_fix(_seg_max(h_jj, ei_rev_jj[0], ei_rev_jj[1], ew_rev_jj, NS))

    out_s = _stage3(xd_req, xd_ss, xd_ss, agg_req, agg_ss, agg_rss,
                    req_Wj, req_bj, ss_Wj, ss_bj, ss_Wj, ss_bj,
                    W_out_s, b_out_s)
    out_j = _stage3(xd_revreq, xd_jj, xd_jj, agg_rreq, agg_jj, agg_rjj,
                    rev_req_Wj, rev_req_bj, jj_Wj, jj_bj, jj_Wj, jj_bj,
                    W_out_j, b_out_j)
    return (out_s, out_j)
